# Initial kernel scaffold; baseline (speedup 1.0000x reference)
import jax
import jax.numpy as jnp
from jax.experimental import pallas as pl


def kernel(x, edge_index, W1, att_src1, att_dst1, b1, W2, att_src2, att_dst2, b2):
    raise NotImplementedError("write your pallas kernel here")


# speedup vs baseline: 53.0107x; 53.0107x over previous
import functools

import jax
import jax.numpy as jnp
from jax import lax
from jax.experimental import pallas as pl
from jax.experimental.pallas import tpu as pltpu
from jax.experimental.pallas import tpu_sc as plsc

N = 10000
E = 320000
D1 = 128
H1 = 8
C1 = 16
D2 = 16

NCORES = 2
NSUB = 16
NTILE = NCORES * NSUB
EPT = E // NTILE
K = 80
NCH = EPT // K
NPAD = 10240
RPT = NPAD // NSUB

BN = 2000
NB = N // BN
BIG = 1e30


def _tc1_body(x_ref, w1_ref, a1s_ref, a1d_ref,
              h_ref, abs_ref, abd_ref, cvec_ref):
    i = pl.program_id(0)
    h = jnp.dot(x_ref[...], w1_ref[...], preferred_element_type=jnp.float32)
    h_ref[...] = h
    s = jnp.dot(h, a1s_ref[...], preferred_element_type=jnp.float32)
    d = jnp.dot(h, a1d_ref[...], preferred_element_type=jnp.float32)
    abs_ref[...] = s
    abd_ref[...] = d
    bm = jnp.max(s, axis=0, keepdims=True)

    @pl.when(i == 0)
    def _():
        cvec_ref[...] = bm

    @pl.when(i > 0)
    def _():
        cvec_ref[...] = jnp.maximum(cvec_ref[...], bm)

    @pl.when(i == NB - 1)
    def _():
        m = cvec_ref[...]
        c = m[:, 0:8] + m[:, 8:16]
        c = jnp.where(c >= 0.0, c, 0.2 * c)
        cvec_ref[...] = jnp.concatenate(
            [c, jnp.full((1, 8), BIG, jnp.float32)], axis=1)


def _tc1(x, W1, A1s, A1d):
    return pl.pallas_call(
        _tc1_body,
        grid=(NB,),
        in_specs=[
            pl.BlockSpec((BN, D1), lambda i: (i, 0)),
            pl.BlockSpec((D1, D1), lambda i: (0, 0)),
            pl.BlockSpec((D1, 16), lambda i: (0, 0)),
            pl.BlockSpec((D1, 16), lambda i: (0, 0)),
        ],
        out_specs=[
            pl.BlockSpec((BN, D1), lambda i: (i, 0)),
            pl.BlockSpec((BN, 16), lambda i: (i, 0)),
            pl.BlockSpec((BN, 16), lambda i: (i, 0)),
            pl.BlockSpec((1, 16), lambda i: (0, 0)),
        ],
        out_shape=[
            jax.ShapeDtypeStruct((N, D1), jnp.float32),
            jax.ShapeDtypeStruct((N, 16), jnp.float32),
            jax.ShapeDtypeStruct((N, 16), jnp.float32),
            jax.ShapeDtypeStruct((1, 16), jnp.float32),
        ],
    )(x, W1, A1s, A1d)


def _tc2_body(nump_ref, denp_ref, b1_ref, w2_ref, p2s_ref, p2d_ref, q_ref,
              h2_ref, abs_ref, abd_ref, cvec_ref, cm_ref):
    i = pl.program_id(0)
    num = nump_ref[0] + nump_ref[1]
    den = denp_ref[0] + denp_ref[1]
    dd = den + 1e-16
    r = 1.0 / dd
    r = r * (2.0 - dd * r)
    r = r * (2.0 - dd * r)
    rb = jnp.dot(r, q_ref[...], preferred_element_type=jnp.float32)
    feat = num * rb + b1_ref[...]
    feat = jnp.where(feat > 0.0, feat, jnp.exp(feat) - 1.0)
    h2 = jnp.dot(feat, w2_ref[...], preferred_element_type=jnp.float32)
    h2_ref[...] = h2
    s = jnp.dot(h2, p2s_ref[...], preferred_element_type=jnp.float32)
    d = jnp.dot(h2, p2d_ref[...], preferred_element_type=jnp.float32)
    abs_ref[...] = s
    abd_ref[...] = d
    bs = jnp.max(s, axis=0, keepdims=True)
    bd = jnp.max(d, axis=0, keepdims=True)

    @pl.when(i == 0)
    def _():
        cm_ref[0:1] = bs
        cm_ref[1:2] = bd

    @pl.when(i > 0)
    def _():
        cm_ref[0:1] = jnp.maximum(cm_ref[0:1], bs)
        cm_ref[1:2] = jnp.maximum(cm_ref[1:2], bd)

    @pl.when(i == NB - 1)
    def _():
        c = cm_ref[0:1] + cm_ref[1:2]
        c = jnp.where(c >= 0.0, c, 0.2 * c)
        lane = lax.broadcasted_iota(jnp.int32, (1, 16), 1)
        cvec_ref[...] = jnp.where(lane == 0, c, BIG)


def _tc2(nump, denp, b1, W2, P2s, P2d, Q16):
    return pl.pallas_call(
        _tc2_body,
        grid=(NB,),
        in_specs=[
            pl.BlockSpec((2, BN, D1), lambda i: (0, i, 0)),
            pl.BlockSpec((2, BN, 16), lambda i: (0, i, 0)),
            pl.BlockSpec((1, D1), lambda i: (0, 0)),
            pl.BlockSpec((D1, D2), lambda i: (0, 0)),
            pl.BlockSpec((D2, 16), lambda i: (0, 0)),
            pl.BlockSpec((D2, 16), lambda i: (0, 0)),
            pl.BlockSpec((16, D1), lambda i: (0, 0)),
        ],
        out_specs=[
            pl.BlockSpec((BN, D2), lambda i: (i, 0)),
            pl.BlockSpec((BN, 16), lambda i: (i, 0)),
            pl.BlockSpec((BN, 16), lambda i: (i, 0)),
            pl.BlockSpec((1, 16), lambda i: (0, 0)),
        ],
        out_shape=[
            jax.ShapeDtypeStruct((N, D2), jnp.float32),
            jax.ShapeDtypeStruct((N, 16), jnp.float32),
            jax.ShapeDtypeStruct((N, 16), jnp.float32),
            jax.ShapeDtypeStruct((1, 16), jnp.float32),
        ],
        scratch_shapes=[pltpu.VMEM((2, 16), jnp.float32)],
    )(nump, denp, b1, W2, P2s, P2d, Q16)


def _tc3_body(nump_ref, denp_ref, b2_ref, out_ref):
    num = nump_ref[0] + nump_ref[1]
    den = denp_ref[0] + denp_ref[1]
    dd = den[:, 0:1] + 1e-16
    r = 1.0 / dd
    r = r * (2.0 - dd * r)
    r = r * (2.0 - dd * r)
    o = num * r + b2_ref[...]
    m = jnp.max(o, axis=1, keepdims=True)
    z = o - m
    lse = jnp.log(jnp.sum(jnp.exp(z), axis=1, keepdims=True))
    out_ref[...] = z - lse


def _tc3(nump, denp, b2):
    return pl.pallas_call(
        _tc3_body,
        grid=(NB,),
        in_specs=[
            pl.BlockSpec((2, BN, 16), lambda i: (0, i, 0)),
            pl.BlockSpec((2, BN, 16), lambda i: (0, i, 0)),
            pl.BlockSpec((1, 16), lambda i: (0, 0)),
        ],
        out_specs=pl.BlockSpec((BN, 16), lambda i: (i, 0)),
        out_shape=jax.ShapeDtypeStruct((N, 16), jnp.float32),
    )(nump, denp, b2)


_MESH = plsc.VectorSubcoreMesh(core_axis_name="c", subcore_axis_name="s")


def _edge1_body(h_hbm, abs_hbm, abd_hbm, esrc_hbm, edst_hbm, cvec_hbm,
                z128_hbm, z16_hbm,
                num_out, den_out,
                num_acc, den_acc, src_v, dst_v, as_r, ad_r, h_r, w_r, cvec_v,
                sem1, sem2, sem3):
    cid = lax.axis_index("c")
    sid = lax.axis_index("s")
    gid = cid * NSUB + sid
    r0 = sid * RPT
    pltpu.sync_copy(z128_hbm.at[pl.ds(r0, RPT)], num_acc.at[pl.ds(r0, RPT)])
    pltpu.sync_copy(z16_hbm.at[pl.ds(r0, RPT)], den_acc.at[pl.ds(r0, RPT)])
    pltpu.sync_copy(cvec_hbm.at[0], cvec_v)
    plsc.subcore_barrier()
    cv = cvec_v[...]

    def chunk(j, carry):
        base = gid * EPT + j * K
        pltpu.sync_copy(esrc_hbm.at[pl.ds(base, K)], src_v)
        pltpu.sync_copy(edst_hbm.at[pl.ds(base, K)], dst_v)
        c1 = pltpu.async_copy(abs_hbm.at[src_v], as_r, sem1)
        c2 = pltpu.async_copy(abd_hbm.at[dst_v], ad_r, sem2)
        c3 = pltpu.async_copy(h_hbm.at[src_v], h_r, sem3)
        c1.wait()
        c2.wait()
        c3.wait()

        def ebody(e, _):
            t = as_r[e] + ad_r[e]
            t = jnp.where(t >= 0.0, t, t * 0.2)
            w = jnp.exp(t - cv)
            w_r[e] = w
            for hh in range(H1):
                ws = w[hh]
                sl = pl.ds(hh * 16, 16)
                h_r[e, sl] = h_r[e, sl] * ws
            return 0

        lax.fori_loop(0, K, ebody, 0)
        pltpu.sync_copy(w_r, den_acc.at[dst_v], add=True)
        pltpu.sync_copy(h_r, num_acc.at[dst_v], add=True)
        return carry

    lax.fori_loop(0, NCH, chunk, 0)
    plsc.subcore_barrier()
    pltpu.sync_copy(num_acc.at[pl.ds(r0, RPT)], num_out.at[cid, pl.ds(r0, RPT)])
    pltpu.sync_copy(den_acc.at[pl.ds(r0, RPT)], den_out.at[cid, pl.ds(r0, RPT)])


_edge1 = functools.partial(
    pl.kernel,
    out_type=[
        jax.ShapeDtypeStruct((NCORES, NPAD, D1), jnp.float32),
        jax.ShapeDtypeStruct((NCORES, NPAD, 16), jnp.float32),
    ],
    mesh=_MESH,
    compiler_params=pltpu.CompilerParams(use_tc_tiling_on_sc=False),
    scratch_types=[
        pltpu.VMEM_SHARED((NPAD, D1), jnp.float32),
        pltpu.VMEM_SHARED((NPAD, 16), jnp.float32),
        pltpu.VMEM((K,), jnp.int32),
        pltpu.VMEM((K,), jnp.int32),
        pltpu.VMEM((K, 16), jnp.float32),
        pltpu.VMEM((K, 16), jnp.float32),
        pltpu.VMEM((K, D1), jnp.float32),
        pltpu.VMEM((K, 16), jnp.float32),
        pltpu.VMEM((16,), jnp.float32),
        pltpu.SemaphoreType.DMA,
        pltpu.SemaphoreType.DMA,
        pltpu.SemaphoreType.DMA,
    ],
)(_edge1_body)


def _edge2_body(h_hbm, abs_hbm, abd_hbm, esrc_hbm, edst_hbm, cvec_hbm,
                z16_hbm,
                num_out, den_out,
                num_acc, den_acc, src_v, dst_v, as_r, ad_r, h_r, w_r, cvec_v,
                sem1, sem2, sem3):
    cid = lax.axis_index("c")
    sid = lax.axis_index("s")
    gid = cid * NSUB + sid
    r0 = sid * RPT
    pltpu.sync_copy(z16_hbm.at[pl.ds(r0, RPT)], num_acc.at[pl.ds(r0, RPT)])
    pltpu.sync_copy(z16_hbm.at[pl.ds(r0, RPT)], den_acc.at[pl.ds(r0, RPT)])
    pltpu.sync_copy(cvec_hbm.at[0], cvec_v)
    plsc.subcore_barrier()
    cv = cvec_v[...]

    def chunk(j, carry):
        base = gid * EPT + j * K
        pltpu.sync_copy(esrc_hbm.at[pl.ds(base, K)], src_v)
        pltpu.sync_copy(edst_hbm.at[pl.ds(base, K)], dst_v)
        c1 = pltpu.async_copy(abs_hbm.at[src_v], as_r, sem1)
        c2 = pltpu.async_copy(abd_hbm.at[dst_v], ad_r, sem2)
        c3 = pltpu.async_copy(h_hbm.at[src_v], h_r, sem3)
        c1.wait()
        c2.wait()
        c3.wait()

        def ebody(e, _):
            t = as_r[e] + ad_r[e]
            t = jnp.where(t >= 0.0, t, t * 0.2)
            w = jnp.exp(t - cv)
            w_r[e] = w
            h_r[e] = h_r[e] * w[0]
            return 0

        lax.fori_loop(0, K, ebody, 0)
        pltpu.sync_copy(w_r, den_acc.at[dst_v], add=True)
        pltpu.sync_copy(h_r, num_acc.at[dst_v], add=True)
        return carry

    lax.fori_loop(0, NCH, chunk, 0)
    plsc.subcore_barrier()
    pltpu.sync_copy(num_acc.at[pl.ds(r0, RPT)], num_out.at[cid, pl.ds(r0, RPT)])
    pltpu.sync_copy(den_acc.at[pl.ds(r0, RPT)], den_out.at[cid, pl.ds(r0, RPT)])


_edge2 = functools.partial(
    pl.kernel,
    out_type=[
        jax.ShapeDtypeStruct((NCORES, NPAD, 16), jnp.float32),
        jax.ShapeDtypeStruct((NCORES, NPAD, 16), jnp.float32),
    ],
    mesh=_MESH,
    compiler_params=pltpu.CompilerParams(use_tc_tiling_on_sc=False),
    scratch_types=[
        pltpu.VMEM_SHARED((NPAD, 16), jnp.float32),
        pltpu.VMEM_SHARED((NPAD, 16), jnp.float32),
        pltpu.VMEM((K,), jnp.int32),
        pltpu.VMEM((K,), jnp.int32),
        pltpu.VMEM((K, 16), jnp.float32),
        pltpu.VMEM((K, 16), jnp.float32),
        pltpu.VMEM((K, 16), jnp.float32),
        pltpu.VMEM((K, 16), jnp.float32),
        pltpu.VMEM((16,), jnp.float32),
        pltpu.SemaphoreType.DMA,
        pltpu.SemaphoreType.DMA,
        pltpu.SemaphoreType.DMA,
    ],
)(_edge2_body)


def kernel(x, edge_index, W1, att_src1, att_dst1, b1, W2, att_src2, att_dst2, b2):
    f32 = jnp.float32
    eye8 = jnp.eye(H1, dtype=f32)
    As = (eye8[:, None, :] * att_src1[:, :, None]).reshape(D1, H1)
    Ad = (eye8[:, None, :] * att_dst1[:, :, None]).reshape(D1, H1)
    A1s = jnp.concatenate([As, Ad], axis=1)
    A1d = jnp.concatenate([Ad, As], axis=1)
    P2s = jnp.concatenate(
        [att_src2.reshape(D2, 1), jnp.zeros((D2, 15), f32)], axis=1)
    P2d = jnp.concatenate(
        [att_dst2.reshape(D2, 1), jnp.zeros((D2, 15), f32)], axis=1)
    Q16 = jnp.concatenate(
        [jnp.kron(jnp.eye(H1, dtype=f32), jnp.ones((1, C1), f32)),
         jnp.zeros((8, D1), f32)], axis=0)
    z128 = jnp.zeros((NPAD, D1), f32)
    z16 = jnp.zeros((NPAD, 16), f32)

    h1, ab1s, ab1d, cvec1 = _tc1(x, W1, A1s, A1d)
    esrc = edge_index[0]
    edst = edge_index[1]
    nump1, denp1 = _edge1(h1, ab1s, ab1d, esrc, edst, cvec1, z128, z16)
    h2, ab2s, ab2d, cvec2 = _tc2(nump1, denp1, b1.reshape(1, D1), W2, P2s,
                                 P2d, Q16)
    nump2, denp2 = _edge2(h2, ab2s, ab2d, esrc, edst, cvec2, z16)
    return _tc3(nump2, denp2, b2.reshape(1, 16))


# speedup vs baseline: 130.4398x; 2.4606x over previous
import functools

import jax
import jax.numpy as jnp
from jax import lax
from jax.experimental import pallas as pl
from jax.experimental.pallas import tpu as pltpu
from jax.experimental.pallas import tpu_sc as plsc

N = 10000
E = 320000
D1 = 128
H1 = 8
C1 = 16
D2 = 16

NCORES = 2
NSUB = 16
NTILE = NCORES * NSUB
EPT = E // NTILE
K = 80
NCH = EPT // K
NPAD = 10240
RPT = NPAD // NSUB

BN = 2000
NB = N // BN
BIG = 1e30


def _tc1_body(x_ref, w1_ref, a1s_ref, a1d_ref,
              h_ref, abs_ref, abd_ref, cvec_ref):
    i = pl.program_id(0)
    h = jnp.dot(x_ref[...], w1_ref[...], preferred_element_type=jnp.float32)
    h_ref[...] = h
    s = jnp.dot(h, a1s_ref[...], preferred_element_type=jnp.float32)
    d = jnp.dot(h, a1d_ref[...], preferred_element_type=jnp.float32)
    abs_ref[...] = s
    abd_ref[...] = d
    bm = jnp.max(s, axis=0, keepdims=True)

    @pl.when(i == 0)
    def _():
        cvec_ref[...] = bm

    @pl.when(i > 0)
    def _():
        cvec_ref[...] = jnp.maximum(cvec_ref[...], bm)

    @pl.when(i == NB - 1)
    def _():
        m = cvec_ref[...]
        c = m[:, 0:8] + m[:, 8:16]
        c = jnp.where(c >= 0.0, c, 0.2 * c)
        cvec_ref[...] = jnp.concatenate(
            [c, jnp.full((1, 8), BIG, jnp.float32)], axis=1)


def _tc1(x, W1, A1s, A1d):
    return pl.pallas_call(
        _tc1_body,
        grid=(NB,),
        in_specs=[
            pl.BlockSpec((BN, D1), lambda i: (i, 0)),
            pl.BlockSpec((D1, D1), lambda i: (0, 0)),
            pl.BlockSpec((D1, 16), lambda i: (0, 0)),
            pl.BlockSpec((D1, 16), lambda i: (0, 0)),
        ],
        out_specs=[
            pl.BlockSpec((BN, D1), lambda i: (i, 0)),
            pl.BlockSpec((BN, 16), lambda i: (i, 0)),
            pl.BlockSpec((BN, 16), lambda i: (i, 0)),
            pl.BlockSpec((1, 16), lambda i: (0, 0)),
        ],
        out_shape=[
            jax.ShapeDtypeStruct((N, D1), jnp.float32),
            jax.ShapeDtypeStruct((N, 16), jnp.float32),
            jax.ShapeDtypeStruct((N, 16), jnp.float32),
            jax.ShapeDtypeStruct((1, 16), jnp.float32),
        ],
    )(x, W1, A1s, A1d)


def _tc2_body(nump_ref, denp_ref, b1_ref, w2_ref, p2s_ref, p2d_ref, q_ref,
              h2_ref, abs_ref, abd_ref, cvec_ref, cm_ref):
    i = pl.program_id(0)
    num = nump_ref[0] + nump_ref[1]
    den = denp_ref[0] + denp_ref[1]
    dd = den + 1e-16
    r = 1.0 / dd
    r = r * (2.0 - dd * r)
    r = r * (2.0 - dd * r)
    rb = jnp.dot(r, q_ref[...], preferred_element_type=jnp.float32)
    feat = num * rb + b1_ref[...]
    feat = jnp.where(feat > 0.0, feat, jnp.exp(feat) - 1.0)
    h2 = jnp.dot(feat, w2_ref[...], preferred_element_type=jnp.float32)
    h2_ref[...] = h2
    s = jnp.dot(h2, p2s_ref[...], preferred_element_type=jnp.float32)
    d = jnp.dot(h2, p2d_ref[...], preferred_element_type=jnp.float32)
    abs_ref[...] = s
    abd_ref[...] = d
    bs = jnp.max(s, axis=0, keepdims=True)
    bd = jnp.max(d, axis=0, keepdims=True)

    @pl.when(i == 0)
    def _():
        cm_ref[0:1] = bs
        cm_ref[1:2] = bd

    @pl.when(i > 0)
    def _():
        cm_ref[0:1] = jnp.maximum(cm_ref[0:1], bs)
        cm_ref[1:2] = jnp.maximum(cm_ref[1:2], bd)

    @pl.when(i == NB - 1)
    def _():
        c = cm_ref[0:1] + cm_ref[1:2]
        c = jnp.where(c >= 0.0, c, 0.2 * c)
        lane = lax.broadcasted_iota(jnp.int32, (1, 16), 1)
        cvec_ref[...] = jnp.where(lane == 0, c, BIG)


def _tc2(nump, denp, b1, W2, P2s, P2d, Q16):
    return pl.pallas_call(
        _tc2_body,
        grid=(NB,),
        in_specs=[
            pl.BlockSpec((2, BN, D1), lambda i: (0, i, 0)),
            pl.BlockSpec((2, BN, 16), lambda i: (0, i, 0)),
            pl.BlockSpec((1, D1), lambda i: (0, 0)),
            pl.BlockSpec((D1, D2), lambda i: (0, 0)),
            pl.BlockSpec((D2, 16), lambda i: (0, 0)),
            pl.BlockSpec((D2, 16), lambda i: (0, 0)),
            pl.BlockSpec((16, D1), lambda i: (0, 0)),
        ],
        out_specs=[
            pl.BlockSpec((BN, D2), lambda i: (i, 0)),
            pl.BlockSpec((BN, 16), lambda i: (i, 0)),
            pl.BlockSpec((BN, 16), lambda i: (i, 0)),
            pl.BlockSpec((1, 16), lambda i: (0, 0)),
        ],
        out_shape=[
            jax.ShapeDtypeStruct((N, D2), jnp.float32),
            jax.ShapeDtypeStruct((N, 16), jnp.float32),
            jax.ShapeDtypeStruct((N, 16), jnp.float32),
            jax.ShapeDtypeStruct((1, 16), jnp.float32),
        ],
        scratch_shapes=[pltpu.VMEM((2, 16), jnp.float32)],
    )(nump, denp, b1, W2, P2s, P2d, Q16)


def _tc3_body(nump_ref, denp_ref, b2_ref, out_ref):
    num = nump_ref[0] + nump_ref[1]
    den = denp_ref[0] + denp_ref[1]
    dd = den[:, 0:1] + 1e-16
    r = 1.0 / dd
    r = r * (2.0 - dd * r)
    r = r * (2.0 - dd * r)
    o = num * r + b2_ref[...]
    m = jnp.max(o, axis=1, keepdims=True)
    z = o - m
    lse = jnp.log(jnp.sum(jnp.exp(z), axis=1, keepdims=True))
    out_ref[...] = z - lse


def _tc3(nump, denp, b2):
    return pl.pallas_call(
        _tc3_body,
        grid=(NB,),
        in_specs=[
            pl.BlockSpec((2, BN, 16), lambda i: (0, i, 0)),
            pl.BlockSpec((2, BN, 16), lambda i: (0, i, 0)),
            pl.BlockSpec((1, 16), lambda i: (0, 0)),
        ],
        out_specs=pl.BlockSpec((BN, 16), lambda i: (i, 0)),
        out_shape=jax.ShapeDtypeStruct((N, 16), jnp.float32),
    )(nump, denp, b2)


_MESH = plsc.VectorSubcoreMesh(core_axis_name="c", subcore_axis_name="s")


def _make_edge_body(D, nheads):
    def body(h_hbm, abs_hbm, abd_hbm, esrc_hbm, dst2d_hbm, cvec_hbm,
             zD_hbm, z16_hbm,
             num_out, den_out,
             num_acc, den_acc, dst_all, si0, si1,
             as0, as1, ad0, ad1, h0, h1, w0, w1, cvec_v,
             ga0, gd0, gh0, ga1, gd1, gh1, sn0, sw0, sn1, sw1, ix0, ix1):
        cid = lax.axis_index("c")
        sid = lax.axis_index("s")
        gid = cid * NSUB + sid
        r0 = sid * RPT
        pltpu.sync_copy(zD_hbm.at[pl.ds(r0, RPT)], num_acc.at[pl.ds(r0, RPT)])
        pltpu.sync_copy(z16_hbm.at[pl.ds(r0, RPT)], den_acc.at[pl.ds(r0, RPT)])
        pltpu.sync_copy(cvec_hbm.at[0], cvec_v)
        pltpu.sync_copy(dst2d_hbm.at[pl.ds(gid * NCH, NCH)], dst_all)
        plsc.subcore_barrier()
        cv = cvec_v[...]

        bufs = ((as0, ad0, h0, w0, si0, ga0, gd0, gh0, sn0, sw0, ix0),
                (as1, ad1, h1, w1, si1, ga1, gd1, gh1, sn1, sw1, ix1))

        def issue_idx(b, j):
            si, ix = bufs[b][4], bufs[b][10]
            pltpu.async_copy(esrc_hbm.at[pl.ds(gid * EPT + j * K, K)], si, ix)

        def wait_idx(b, j):
            si, ix = bufs[b][4], bufs[b][10]
            pltpu.make_async_copy(
                esrc_hbm.at[pl.ds(gid * EPT + j * K, K)], si, ix).wait()

        def issue_g(b, j):
            as_r, ad_r, h_r, _w, si, ga, gd, gh = bufs[b][:8]
            pltpu.async_copy(abs_hbm.at[si], as_r, ga)
            pltpu.async_copy(abd_hbm.at[dst_all.at[j]], ad_r, gd)
            pltpu.async_copy(h_hbm.at[si], h_r, gh)

        def wait_g(b, j):
            as_r, ad_r, h_r, _w, si, ga, gd, gh = bufs[b][:8]
            pltpu.make_async_copy(abs_hbm.at[si], as_r, ga).wait()
            pltpu.make_async_copy(abd_hbm.at[dst_all.at[j]], ad_r, gd).wait()
            pltpu.make_async_copy(h_hbm.at[si], h_r, gh).wait()

        def issue_s(b, j):
            h_r, w_r, sn, sw = bufs[b][2], bufs[b][3], bufs[b][8], bufs[b][9]
            pltpu.async_copy(h_r, num_acc.at[dst_all.at[j]], sn, add=True)
            pltpu.async_copy(w_r, den_acc.at[dst_all.at[j]], sw, add=True)

        def wait_s(b, j):
            h_r, w_r, sn, sw = bufs[b][2], bufs[b][3], bufs[b][8], bufs[b][9]
            pltpu.make_async_copy(h_r, num_acc.at[dst_all.at[j]], sn).wait()
            pltpu.make_async_copy(w_r, den_acc.at[dst_all.at[j]], sw).wait()

        def compute(b):
            as_r, ad_r, h_r, w_r = bufs[b][:4]

            @plsc.parallel_loop(0, K, 1, unroll=2)
            def _(e):
                t = as_r[e] + ad_r[e]
                t = jnp.where(t >= 0.0, t, t * 0.2)
                w = jnp.exp(t - cv)
                w_r[e] = w
                if nheads == 1:
                    h_r[e] = h_r[e] * w[0]
                else:
                    for hh in range(nheads):
                        sl = pl.ds(hh * 16, 16)
                        h_r[e, sl] = h_r[e, sl] * w[hh]

        issue_idx(0, 0)
        issue_idx(1, 1)
        wait_idx(0, 0)
        issue_g(0, 0)
        wait_idx(1, 1)
        issue_g(1, 1)

        def pair(jj, carry):
            wait_g(0, jj)

            @pl.when(jj + 2 < NCH)
            def _():
                issue_idx(0, jj + 2)

            compute(0)
            issue_s(0, jj)
            wait_g(1, jj + 1)

            @pl.when(jj + 3 < NCH)
            def _():
                issue_idx(1, jj + 3)

            compute(1)
            issue_s(1, jj + 1)

            @pl.when(jj + 2 < NCH)
            def _():
                wait_s(0, jj)
                wait_idx(0, jj + 2)
                issue_g(0, jj + 2)

            @pl.when(jj + 3 < NCH)
            def _():
                wait_s(1, jj + 1)
                wait_idx(1, jj + 3)
                issue_g(1, jj + 3)

            return carry

        lax.fori_loop(0, (NCH - 1) // 2, lambda i, c: pair(2 * i, c), 0)
        wait_g(0, NCH - 1)
        compute(0)
        issue_s(0, NCH - 1)
        wait_s(0, NCH - 1)
        wait_s(1, NCH - 2)
        plsc.subcore_barrier()
        pltpu.sync_copy(num_acc.at[pl.ds(r0, RPT)],
                        num_out.at[cid, pl.ds(r0, RPT)])
        pltpu.sync_copy(den_acc.at[pl.ds(r0, RPT)],
                        den_out.at[cid, pl.ds(r0, RPT)])

    return body


def _make_edge(D, nheads):
    sems = [pltpu.SemaphoreType.DMA] * 12
    return functools.partial(
        pl.kernel,
        out_type=[
            jax.ShapeDtypeStruct((NCORES, NPAD, D), jnp.float32),
            jax.ShapeDtypeStruct((NCORES, NPAD, 16), jnp.float32),
        ],
        mesh=_MESH,
        compiler_params=pltpu.CompilerParams(use_tc_tiling_on_sc=False),
        scratch_types=[
            pltpu.VMEM_SHARED((NPAD, D), jnp.float32),
            pltpu.VMEM_SHARED((NPAD, 16), jnp.float32),
            pltpu.VMEM((NCH, K), jnp.int32),
            pltpu.VMEM((K,), jnp.int32),
            pltpu.VMEM((K,), jnp.int32),
            pltpu.VMEM((K, 16), jnp.float32),
            pltpu.VMEM((K, 16), jnp.float32),
            pltpu.VMEM((K, 16), jnp.float32),
            pltpu.VMEM((K, 16), jnp.float32),
            pltpu.VMEM((K, D), jnp.float32),
            pltpu.VMEM((K, D), jnp.float32),
            pltpu.VMEM((K, 16), jnp.float32),
            pltpu.VMEM((K, 16), jnp.float32),
            pltpu.VMEM((16,), jnp.float32),
        ] + sems,
    )(_make_edge_body(D, nheads))


_edge1 = _make_edge(D1, H1)
_edge2 = _make_edge(D2, 1)


def kernel(x, edge_index, W1, att_src1, att_dst1, b1, W2, att_src2, att_dst2, b2):
    f32 = jnp.float32
    eye8 = jnp.eye(H1, dtype=f32)
    As = (eye8[:, None, :] * att_src1[:, :, None]).reshape(D1, H1)
    Ad = (eye8[:, None, :] * att_dst1[:, :, None]).reshape(D1, H1)
    A1s = jnp.concatenate([As, Ad], axis=1)
    A1d = jnp.concatenate([Ad, As], axis=1)
    P2s = jnp.concatenate(
        [att_src2.reshape(D2, 1), jnp.zeros((D2, 15), f32)], axis=1)
    P2d = jnp.concatenate(
        [att_dst2.reshape(D2, 1), jnp.zeros((D2, 15), f32)], axis=1)
    Q16 = jnp.concatenate(
        [jnp.kron(jnp.eye(H1, dtype=f32), jnp.ones((1, C1), f32)),
         jnp.zeros((8, D1), f32)], axis=0)
    z128 = jnp.zeros((NPAD, D1), f32)
    z16 = jnp.zeros((NPAD, 16), f32)

    h1, ab1s, ab1d, cvec1 = _tc1(x, W1, A1s, A1d)
    esrc = edge_index[0]
    dst2d = edge_index[1].reshape(E // K, K)
    nump1, denp1 = _edge1(h1, ab1s, ab1d, esrc, dst2d, cvec1, z128, z16)
    h2, ab2s, ab2d, cvec2 = _tc2(nump1, denp1, b1.reshape(1, D1), W2, P2s,
                                 P2d, Q16)
    nump2, denp2 = _edge2(h2, ab2s, ab2d, esrc, dst2d, cvec2, z16, z16)
    return _tc3(nump2, denp2, b2.reshape(1, 16))


# speedup vs baseline: 130.5185x; 1.0006x over previous
import functools

import jax
import jax.numpy as jnp
from jax import lax
from jax.experimental import pallas as pl
from jax.experimental.pallas import tpu as pltpu
from jax.experimental.pallas import tpu_sc as plsc

N = 10000
E = 320000
D1 = 128
H1 = 8
C1 = 16
D2 = 16

NCORES = 2
NSUB = 16
NTILE = NCORES * NSUB
EPT = E // NTILE
K = 80
NCH = EPT // K
NPAD = 10240
RPT = NPAD // NSUB

BN = 2000
NB = N // BN
BIG = 1e30


def _tc1_body(x_ref, w1_ref, a1s_ref, a1d_ref,
              h_ref, abs_ref, abd_ref, cvec_ref):
    i = pl.program_id(0)
    h = jnp.dot(x_ref[...], w1_ref[...], preferred_element_type=jnp.float32)
    h_ref[...] = h
    s = jnp.dot(h, a1s_ref[...], preferred_element_type=jnp.float32)
    d = jnp.dot(h, a1d_ref[...], preferred_element_type=jnp.float32)
    abs_ref[...] = s
    abd_ref[...] = d
    bm = jnp.max(s, axis=0, keepdims=True)

    @pl.when(i == 0)
    def _():
        cvec_ref[...] = bm

    @pl.when(i > 0)
    def _():
        cvec_ref[...] = jnp.maximum(cvec_ref[...], bm)

    @pl.when(i == NB - 1)
    def _():
        m = cvec_ref[...]
        c = m[:, 0:8] + m[:, 8:16]
        c = jnp.where(c >= 0.0, c, 0.2 * c)
        cvec_ref[...] = jnp.concatenate(
            [c, jnp.full((1, 8), BIG, jnp.float32)], axis=1)


def _tc1(x, W1, A1s, A1d):
    return pl.pallas_call(
        _tc1_body,
        grid=(NB,),
        in_specs=[
            pl.BlockSpec((BN, D1), lambda i: (i, 0)),
            pl.BlockSpec((D1, D1), lambda i: (0, 0)),
            pl.BlockSpec((D1, 16), lambda i: (0, 0)),
            pl.BlockSpec((D1, 16), lambda i: (0, 0)),
        ],
        out_specs=[
            pl.BlockSpec((BN, D1), lambda i: (i, 0)),
            pl.BlockSpec((BN, 16), lambda i: (i, 0)),
            pl.BlockSpec((BN, 16), lambda i: (i, 0)),
            pl.BlockSpec((1, 16), lambda i: (0, 0)),
        ],
        out_shape=[
            jax.ShapeDtypeStruct((N, D1), jnp.float32),
            jax.ShapeDtypeStruct((N, 16), jnp.float32),
            jax.ShapeDtypeStruct((N, 16), jnp.float32),
            jax.ShapeDtypeStruct((1, 16), jnp.float32),
        ],
    )(x, W1, A1s, A1d)


def _tc2_body(nump_ref, denp_ref, b1_ref, w2_ref, p2s_ref, p2d_ref, q_ref,
              h2_ref, abs_ref, abd_ref, cvec_ref, cm_ref):
    i = pl.program_id(0)
    num = nump_ref[0] + nump_ref[1]
    den = denp_ref[0] + denp_ref[1]
    dd = den + 1e-16
    r = 1.0 / dd
    r = r * (2.0 - dd * r)
    r = r * (2.0 - dd * r)
    rb = jnp.dot(r, q_ref[...], preferred_element_type=jnp.float32)
    feat = num * rb + b1_ref[...]
    feat = jnp.where(feat > 0.0, feat, jnp.exp(feat) - 1.0)
    h2 = jnp.dot(feat, w2_ref[...], preferred_element_type=jnp.float32)
    h2_ref[...] = h2
    s = jnp.dot(h2, p2s_ref[...], preferred_element_type=jnp.float32)
    d = jnp.dot(h2, p2d_ref[...], preferred_element_type=jnp.float32)
    abs_ref[...] = s
    abd_ref[...] = d
    bs = jnp.max(s, axis=0, keepdims=True)
    bd = jnp.max(d, axis=0, keepdims=True)

    @pl.when(i == 0)
    def _():
        cm_ref[0:1] = bs
        cm_ref[1:2] = bd

    @pl.when(i > 0)
    def _():
        cm_ref[0:1] = jnp.maximum(cm_ref[0:1], bs)
        cm_ref[1:2] = jnp.maximum(cm_ref[1:2], bd)

    @pl.when(i == NB - 1)
    def _():
        c = cm_ref[0:1] + cm_ref[1:2]
        c = jnp.where(c >= 0.0, c, 0.2 * c)
        lane = lax.broadcasted_iota(jnp.int32, (1, 16), 1)
        cvec_ref[...] = jnp.where(lane == 0, c, BIG)


def _tc2(nump, denp, b1, W2, P2s, P2d, Q16):
    return pl.pallas_call(
        _tc2_body,
        grid=(NB,),
        in_specs=[
            pl.BlockSpec((2, BN, D1), lambda i: (0, i, 0)),
            pl.BlockSpec((2, BN, 16), lambda i: (0, i, 0)),
            pl.BlockSpec((1, D1), lambda i: (0, 0)),
            pl.BlockSpec((D1, D2), lambda i: (0, 0)),
            pl.BlockSpec((D2, 16), lambda i: (0, 0)),
            pl.BlockSpec((D2, 16), lambda i: (0, 0)),
            pl.BlockSpec((16, D1), lambda i: (0, 0)),
        ],
        out_specs=[
            pl.BlockSpec((BN, D2), lambda i: (i, 0)),
            pl.BlockSpec((BN, 16), lambda i: (i, 0)),
            pl.BlockSpec((BN, 16), lambda i: (i, 0)),
            pl.BlockSpec((1, 16), lambda i: (0, 0)),
        ],
        out_shape=[
            jax.ShapeDtypeStruct((N, D2), jnp.float32),
            jax.ShapeDtypeStruct((N, 16), jnp.float32),
            jax.ShapeDtypeStruct((N, 16), jnp.float32),
            jax.ShapeDtypeStruct((1, 16), jnp.float32),
        ],
        scratch_shapes=[pltpu.VMEM((2, 16), jnp.float32)],
    )(nump, denp, b1, W2, P2s, P2d, Q16)


def _tc3_body(nump_ref, denp_ref, b2_ref, out_ref):
    num = nump_ref[0] + nump_ref[1]
    den = denp_ref[0] + denp_ref[1]
    dd = den[:, 0:1] + 1e-16
    r = 1.0 / dd
    r = r * (2.0 - dd * r)
    r = r * (2.0 - dd * r)
    o = num * r + b2_ref[...]
    m = jnp.max(o, axis=1, keepdims=True)
    z = o - m
    lse = jnp.log(jnp.sum(jnp.exp(z), axis=1, keepdims=True))
    out_ref[...] = z - lse


def _tc3(nump, denp, b2):
    return pl.pallas_call(
        _tc3_body,
        grid=(NB,),
        in_specs=[
            pl.BlockSpec((2, BN, 16), lambda i: (0, i, 0)),
            pl.BlockSpec((2, BN, 16), lambda i: (0, i, 0)),
            pl.BlockSpec((1, 16), lambda i: (0, 0)),
        ],
        out_specs=pl.BlockSpec((BN, 16), lambda i: (i, 0)),
        out_shape=jax.ShapeDtypeStruct((N, 16), jnp.float32),
    )(nump, denp, b2)


_MESH = plsc.VectorSubcoreMesh(core_axis_name="c", subcore_axis_name="s")


def _make_edge_body(D, nheads):
    def body(h_hbm, abs_hbm, abd_hbm, esrc_hbm, dst2d_hbm, cvec_hbm,
             zD_hbm, z16_hbm,
             num_out, den_out,
             num_acc, den_acc, dst_all, si0, si1,
             as0, as1, ad0, ad1, h0, h1, w0, w1, cvec_v,
             ga0, gd0, gh0, ga1, gd1, gh1, sn0, sw0, sn1, sw1, ix0, ix1):
        cid = lax.axis_index("c")
        sid = lax.axis_index("s")
        gid = cid * NSUB + sid
        r0 = sid * RPT
        pltpu.sync_copy(zD_hbm.at[pl.ds(r0, RPT)], num_acc.at[pl.ds(r0, RPT)])
        pltpu.sync_copy(z16_hbm.at[pl.ds(r0, RPT)], den_acc.at[pl.ds(r0, RPT)])
        pltpu.sync_copy(cvec_hbm.at[0], cvec_v)
        pltpu.sync_copy(dst2d_hbm.at[pl.ds(gid * NCH, NCH)], dst_all)
        plsc.subcore_barrier()
        cv = cvec_v[...]

        bufs = ((as0, ad0, h0, w0, si0, ga0, gd0, gh0, sn0, sw0, ix0),
                (as1, ad1, h1, w1, si1, ga1, gd1, gh1, sn1, sw1, ix1))

        def issue_idx(b, j):
            si, ix = bufs[b][4], bufs[b][10]
            pltpu.async_copy(esrc_hbm.at[pl.ds(gid * EPT + j * K, K)], si, ix)

        def wait_idx(b, j):
            si, ix = bufs[b][4], bufs[b][10]
            pltpu.make_async_copy(
                esrc_hbm.at[pl.ds(gid * EPT + j * K, K)], si, ix).wait()

        def issue_g(b, j):
            as_r, ad_r, h_r, _w, si, ga, gd, gh = bufs[b][:8]
            pltpu.async_copy(abs_hbm.at[si], as_r, ga)
            pltpu.async_copy(abd_hbm.at[dst_all.at[j]], ad_r, gd)
            pltpu.async_copy(h_hbm.at[si], h_r, gh)

        def wait_g(b, j):
            as_r, ad_r, h_r, _w, si, ga, gd, gh = bufs[b][:8]
            pltpu.make_async_copy(abs_hbm.at[si], as_r, ga).wait()
            pltpu.make_async_copy(abd_hbm.at[dst_all.at[j]], ad_r, gd).wait()
            pltpu.make_async_copy(h_hbm.at[si], h_r, gh).wait()

        def issue_s(b, j):
            h_r, w_r, sn, sw = bufs[b][2], bufs[b][3], bufs[b][8], bufs[b][9]
            pltpu.async_copy(h_r, num_acc.at[dst_all.at[j]], sn, add=True)
            pltpu.async_copy(w_r, den_acc.at[dst_all.at[j]], sw, add=True)

        def wait_s(b, j):
            h_r, w_r, sn, sw = bufs[b][2], bufs[b][3], bufs[b][8], bufs[b][9]
            pltpu.make_async_copy(h_r, num_acc.at[dst_all.at[j]], sn).wait()
            pltpu.make_async_copy(w_r, den_acc.at[dst_all.at[j]], sw).wait()

        def compute(b):
            as_r, ad_r, h_r, w_r = bufs[b][:4]

            @plsc.parallel_loop(0, K, 1, unroll=4)
            def _(e):
                t = as_r[e] + ad_r[e]
                t = jnp.where(t >= 0.0, t, t * 0.2)
                w = jnp.exp(t - cv)
                w_r[e] = w
                if nheads == 1:
                    h_r[e] = h_r[e] * w[0]
                else:
                    for hh in range(nheads):
                        sl = pl.ds(hh * 16, 16)
                        h_r[e, sl] = h_r[e, sl] * w[hh]

        issue_idx(0, 0)
        issue_idx(1, 1)
        wait_idx(0, 0)
        issue_g(0, 0)
        wait_idx(1, 1)
        issue_g(1, 1)

        def pair(jj, carry):
            wait_g(0, jj)

            @pl.when(jj + 2 < NCH)
            def _():
                issue_idx(0, jj + 2)

            compute(0)
            issue_s(0, jj)
            wait_g(1, jj + 1)

            @pl.when(jj + 3 < NCH)
            def _():
                issue_idx(1, jj + 3)

            compute(1)
            issue_s(1, jj + 1)

            @pl.when(jj + 2 < NCH)
            def _():
                wait_s(0, jj)
                wait_idx(0, jj + 2)
                issue_g(0, jj + 2)

            @pl.when(jj + 3 < NCH)
            def _():
                wait_s(1, jj + 1)
                wait_idx(1, jj + 3)
                issue_g(1, jj + 3)

            return carry

        lax.fori_loop(0, (NCH - 1) // 2, lambda i, c: pair(2 * i, c), 0)
        wait_g(0, NCH - 1)
        compute(0)
        issue_s(0, NCH - 1)
        wait_s(0, NCH - 1)
        wait_s(1, NCH - 2)
        plsc.subcore_barrier()
        pltpu.sync_copy(num_acc.at[pl.ds(r0, RPT)],
                        num_out.at[cid, pl.ds(r0, RPT)])
        pltpu.sync_copy(den_acc.at[pl.ds(r0, RPT)],
                        den_out.at[cid, pl.ds(r0, RPT)])

    return body


def _make_edge(D, nheads):
    sems = [pltpu.SemaphoreType.DMA] * 12
    return functools.partial(
        pl.kernel,
        out_type=[
            jax.ShapeDtypeStruct((NCORES, NPAD, D), jnp.float32),
            jax.ShapeDtypeStruct((NCORES, NPAD, 16), jnp.float32),
        ],
        mesh=_MESH,
        compiler_params=pltpu.CompilerParams(use_tc_tiling_on_sc=False),
        scratch_types=[
            pltpu.VMEM_SHARED((NPAD, D), jnp.float32),
            pltpu.VMEM_SHARED((NPAD, 16), jnp.float32),
            pltpu.VMEM((NCH, K), jnp.int32),
            pltpu.VMEM((K,), jnp.int32),
            pltpu.VMEM((K,), jnp.int32),
            pltpu.VMEM((K, 16), jnp.float32),
            pltpu.VMEM((K, 16), jnp.float32),
            pltpu.VMEM((K, 16), jnp.float32),
            pltpu.VMEM((K, 16), jnp.float32),
            pltpu.VMEM((K, D), jnp.float32),
            pltpu.VMEM((K, D), jnp.float32),
            pltpu.VMEM((K, 16), jnp.float32),
            pltpu.VMEM((K, 16), jnp.float32),
            pltpu.VMEM((16,), jnp.float32),
        ] + sems,
    )(_make_edge_body(D, nheads))


_edge1 = _make_edge(D1, H1)
_edge2 = _make_edge(D2, 1)


def kernel(x, edge_index, W1, att_src1, att_dst1, b1, W2, att_src2, att_dst2, b2):
    f32 = jnp.float32
    eye8 = jnp.eye(H1, dtype=f32)
    As = (eye8[:, None, :] * att_src1[:, :, None]).reshape(D1, H1)
    Ad = (eye8[:, None, :] * att_dst1[:, :, None]).reshape(D1, H1)
    A1s = jnp.concatenate([As, Ad], axis=1)
    A1d = jnp.concatenate([Ad, As], axis=1)
    P2s = jnp.concatenate(
        [att_src2.reshape(D2, 1), jnp.zeros((D2, 15), f32)], axis=1)
    P2d = jnp.concatenate(
        [att_dst2.reshape(D2, 1), jnp.zeros((D2, 15), f32)], axis=1)
    Q16 = jnp.concatenate(
        [jnp.kron(jnp.eye(H1, dtype=f32), jnp.ones((1, C1), f32)),
         jnp.zeros((8, D1), f32)], axis=0)
    z128 = jnp.zeros((NPAD, D1), f32)
    z16 = jnp.zeros((NPAD, 16), f32)

    h1, ab1s, ab1d, cvec1 = _tc1(x, W1, A1s, A1d)
    esrc = edge_index[0]
    dst2d = edge_index[1].reshape(E // K, K)
    nump1, denp1 = _edge1(h1, ab1s, ab1d, esrc, dst2d, cvec1, z128, z16)
    h2, ab2s, ab2d, cvec2 = _tc2(nump1, denp1, b1.reshape(1, D1), W2, P2s,
                                 P2d, Q16)
    nump2, denp2 = _edge2(h2, ab2s, ab2d, esrc, dst2d, cvec2, z16, z16)
    return _tc3(nump2, denp2, b2.reshape(1, 16))
